# baseline (speedup 1.0000x reference)
import jax
import jax.numpy as jnp
from jax.experimental import pallas as pl

_SZ = 2048
_TEMP = 10.0
_EPS = 1e-20
_NROW = _SZ * _SZ // 64
_BS = 2048


def _body(a_ref, u_ref, o_ref):
    l_idx = jax.lax.broadcasted_iota(jnp.int32, (128, 64), 0)
    k_idx = jax.lax.broadcasted_iota(jnp.int32, (128, 64), 1)
    d = jnp.where(l_idx == 2 * k_idx, 1.0, 0.0) - jnp.where(
        l_idx == 2 * k_idx + 1, 1.0, 0.0
    )
    w = (a_ref[...] - jnp.log(_EPS - jnp.log(u_ref[...] + _EPS))) * (1.0 / _TEMP)
    x = jax.lax.dot_general(
        w,
        d,
        (((1,), (0,)), ((), ())),
        precision=jax.lax.Precision.HIGHEST,
        preferred_element_type=jnp.float32,
    )
    o_ref[...] = jax.nn.sigmoid(x)


def kernel(gen_matrix, u):
    a = gen_matrix.reshape(_NROW, 128)
    uu = u.reshape(_NROW, 128)
    out = pl.pallas_call(
        _body,
        grid=(_NROW // _BS,),
        in_specs=[
            pl.BlockSpec((_BS, 128), lambda i: (i, 0)),
            pl.BlockSpec((_BS, 128), lambda i: (i, 0)),
        ],
        out_specs=pl.BlockSpec((_BS, 64), lambda i: (i, 0)),
        out_shape=jax.ShapeDtypeStruct((_NROW, 64), jnp.float32),
    )(a, uu)
    return out.reshape(_SZ, _SZ)


# speedup vs baseline: 65.5586x; 65.5586x over previous
import jax
import jax.numpy as jnp
from jax.experimental import pallas as pl

_SZ = 2048
_TEMP = 10.0
_EPS = 1e-20
_BR = 256


def _body(a0_ref, a1_ref, u0_ref, u1_ref, o_ref):
    l0 = _EPS - jnp.log(u0_ref[...] + _EPS)
    l1 = _EPS - jnp.log(u1_ref[...] + _EPS)
    x = (a0_ref[...] - a1_ref[...] + jnp.log(l1 / l0)) * (1.0 / _TEMP)
    o_ref[...] = jax.nn.sigmoid(x)


def kernel(gen_matrix, u):
    u3 = u.reshape(_SZ, _SZ, 2)
    a0 = gen_matrix[:, :, 0]
    a1 = gen_matrix[:, :, 1]
    u0 = u3[:, :, 0]
    u1 = u3[:, :, 1]
    spec = pl.BlockSpec((_BR, _SZ), lambda i: (i, 0))
    return pl.pallas_call(
        _body,
        grid=(_SZ // _BR,),
        in_specs=[spec, spec, spec, spec],
        out_specs=spec,
        out_shape=jax.ShapeDtypeStruct((_SZ, _SZ), jnp.float32),
    )(a0, a1, u0, u1)
